# Initial kernel scaffold; baseline (speedup 1.0000x reference)
import jax
import jax.numpy as jnp
from jax.experimental import pallas as pl


def kernel(x, b_pre, W_enc, b_enc, W_dec, b_dec):
    raise NotImplementedError("write your pallas kernel here")


# speedup vs baseline: 5.2802x; 5.2802x over previous
import functools

import jax
import jax.numpy as jnp
from jax.experimental import pallas as pl

_K = 32


def _enc_body(x_ref, bpre_ref, w_ref, benc_ref, out_ref):
    xc = x_ref[...] - bpre_ref[...]
    acc = jax.lax.dot_general(
        xc, w_ref[...],
        dimension_numbers=(((1,), (1,)), ((), ())),
        precision=jax.lax.Precision.DEFAULT,
        preferred_element_type=jnp.float32,
    )
    out_ref[...] = acc + benc_ref[...]


def _topk_body(pre_ref, hid_ref, l0_ref, *, k):
    t = pl.program_id(0)
    v = pre_ref[...]
    u = jax.lax.bitcast_convert_type(v, jnp.int32)
    key = u ^ (jnp.right_shift(u, 31) & jnp.int32(0x7FFFFFFF))

    rows = v.shape[0]
    lo = jnp.full((rows, 1), jnp.iinfo(jnp.int32).min, jnp.int32)
    hi = jnp.full((rows, 1), jnp.iinfo(jnp.int32).max, jnp.int32)

    def step(_, carry):
        lo, hi = carry
        xo = lo ^ hi
        mid = (lo & hi) + jnp.right_shift(xo, 1) + (xo & 1)
        cnt = jnp.sum((key >= mid).astype(jnp.int32), axis=1, keepdims=True)
        p = cnt >= k
        return jnp.where(p, mid, lo), jnp.where(p, hi, mid - 1)

    lo, hi = jax.lax.fori_loop(0, 33, step, (lo, hi))
    mask = key >= lo
    hid_ref[...] = jnp.where(mask, jnp.maximum(v, 0.0), 0.0)

    @pl.when(t == 0)
    def _():
        l0_ref[...] = jnp.zeros_like(l0_ref)

    pos = jnp.sum((mask & (v > 0.0)).astype(jnp.float32))
    l0_ref[...] += jnp.full((1, 1), pos, jnp.float32)


def _dec_body(hid_ref, w_ref, x_ref, bdec_ref, bpre_ref, out_ref, loss_ref):
    t = pl.program_id(0)
    kk = pl.program_id(1)
    nk = pl.num_programs(1)
    part = jax.lax.dot_general(
        hid_ref[...], w_ref[...],
        dimension_numbers=(((1,), (1,)), ((), ())),
        precision=jax.lax.Precision.DEFAULT,
        preferred_element_type=jnp.float32,
    )

    @pl.when(kk == 0)
    def _():
        out_ref[...] = part

    @pl.when(kk > 0)
    def _():
        out_ref[...] += part

    @pl.when((t == 0) & (kk == 0))
    def _():
        loss_ref[...] = jnp.zeros_like(loss_ref)

    @pl.when(kk == nk - 1)
    def _():
        total = out_ref[...] + bdec_ref[...] + bpre_ref[...]
        out_ref[...] = total
        d = total - x_ref[...]
        loss_ref[...] += jnp.full((1, 1), jnp.sum(d * d), jnp.float32)


def kernel(x, b_pre, W_enc, b_enc, W_dec, b_dec):
    n, d = x.shape
    h = W_enc.shape[0]
    f32 = jnp.float32

    bpre2 = b_pre.reshape(1, d)
    benc2 = b_enc.reshape(1, h)
    bdec2 = b_dec.reshape(1, d)

    tb = min(1024, n)
    hb = min(512, h)
    pre = pl.pallas_call(
        _enc_body,
        grid=(n // tb, h // hb),
        in_specs=[
            pl.BlockSpec((tb, d), lambda t, hh: (t, 0)),
            pl.BlockSpec((1, d), lambda t, hh: (0, 0)),
            pl.BlockSpec((hb, d), lambda t, hh: (hh, 0)),
            pl.BlockSpec((1, hb), lambda t, hh: (0, hh)),
        ],
        out_specs=pl.BlockSpec((tb, hb), lambda t, hh: (t, hh)),
        out_shape=jax.ShapeDtypeStruct((n, h), f32),
    )(x, bpre2, W_enc, benc2)

    rb = min(128, n)
    hidden, l0_sum = pl.pallas_call(
        functools.partial(_topk_body, k=_K),
        grid=(n // rb,),
        in_specs=[pl.BlockSpec((rb, h), lambda t: (t, 0))],
        out_specs=[
            pl.BlockSpec((rb, h), lambda t: (t, 0)),
            pl.BlockSpec((1, 1), lambda t: (0, 0)),
        ],
        out_shape=[
            jax.ShapeDtypeStruct((n, h), f32),
            jax.ShapeDtypeStruct((1, 1), f32),
        ],
        input_output_aliases={0: 0},
    )(pre)

    td = min(512, n)
    kb = min(1024, h)
    recon, loss_sum = pl.pallas_call(
        _dec_body,
        grid=(n // td, h // kb),
        in_specs=[
            pl.BlockSpec((td, kb), lambda t, kk: (t, kk)),
            pl.BlockSpec((d, kb), lambda t, kk: (0, kk)),
            pl.BlockSpec((td, d), lambda t, kk: (t, 0)),
            pl.BlockSpec((1, d), lambda t, kk: (0, 0)),
            pl.BlockSpec((1, d), lambda t, kk: (0, 0)),
        ],
        out_specs=[
            pl.BlockSpec((td, d), lambda t, kk: (t, 0)),
            pl.BlockSpec((1, 1), lambda t, kk: (0, 0)),
        ],
        out_shape=[
            jax.ShapeDtypeStruct((n, d), f32),
            jax.ShapeDtypeStruct((1, 1), f32),
        ],
    )(hidden, W_dec, x, bdec2, bpre2)

    rec_loss = loss_sum[0, 0] / jnp.float32(n * d)
    l0 = l0_sum[0, 0] / jnp.float32(n)
    sparsity = jnp.zeros((), f32)
    return (recon, hidden, rec_loss, rec_loss, sparsity, l0)
